# Initial kernel scaffold; baseline (speedup 1.0000x reference)
import jax
import jax.numpy as jnp
from jax.experimental import pallas as pl


def kernel(x, position_ids):
    raise NotImplementedError("write your pallas kernel here")


# speedup vs baseline: 1.2405x; 1.2405x over previous
import functools
import math

import numpy as np
import jax
import jax.numpy as jnp
from jax import lax
from jax.experimental import pallas as pl
from jax.experimental.pallas import tpu as pltpu
from jax.experimental.pallas import tpu_sc as plsc

DIM = 128
BASE = 10000.0
SCALING = 4.0
ORIG_MAX = 8192
BETA_FAST = 32.0
BETA_SLOW = 1.0
ATTN_FACTOR = 1.0
MAX_POS = 32768


def _yarn_correction_dim(num_rotations):
    return DIM * math.log(ORIG_MAX / (num_rotations * 2 * math.pi)) / (2 * math.log(BASE))


def _build_tables():
    inv_freq_base = 1.0 / (BASE ** (np.arange(0, DIM, 2, dtype=np.float32) / DIM))
    inv_freq_interp = inv_freq_base / SCALING
    low = max(math.floor(_yarn_correction_dim(BETA_FAST)), 0)
    high = min(math.ceil(_yarn_correction_dim(BETA_SLOW)), DIM - 1)
    mn, mx = float(low), float(high)
    if mn == mx:
        mx += 0.001
    ramp = np.clip((np.arange(DIM // 2, dtype=np.float32) - mn) / (mx - mn), 0.0, 1.0)
    inv_freq_mask = 1.0 - ramp
    inv_freq = inv_freq_interp * (1 - inv_freq_mask) + inv_freq_base * inv_freq_mask
    t = np.arange(MAX_POS, dtype=np.float32)
    freqs = np.outer(t, inv_freq)
    emb = np.concatenate((freqs, freqs), axis=-1)
    cos_tab = (np.cos(emb) * ATTN_FACTOR).astype(np.float32)
    sin_tab = (np.sin(emb) * ATTN_FACTOR).astype(np.float32)
    return cos_tab, sin_tab


_COS_TAB, _SIN_TAB = _build_tables()

_INFO = plsc.get_sparse_core_info()
_NC, _NS = _INFO.num_cores, _INFO.num_subcores
_NW = _NC * _NS

_B = 16384
_CHUNK = 128
_NCHUNK = _B // (_NW * _CHUNK)


def _gather_body(cos_hbm, sin_hbm, idx_hbm, cos_out, sin_out,
                 idx_v, cos_buf, sin_buf, sem_c, sem_s):
    wid = lax.axis_index("s") * _NC + lax.axis_index("c")
    pltpu.sync_copy(idx_hbm.at[pl.ds(wid * _NCHUNK, _NCHUNK)], idx_v)
    base = wid * _NCHUNK * _CHUNK
    for j in range(_NCHUNK):
        cp_c = pltpu.async_copy(cos_hbm.at[idx_v.at[j]], cos_buf, sem_c)
        cp_s = pltpu.async_copy(sin_hbm.at[idx_v.at[j]], sin_buf, sem_s)
        cp_c.wait()
        pltpu.sync_copy(cos_buf, cos_out.at[pl.ds(base + j * _CHUNK, _CHUNK)])
        cp_s.wait()
        pltpu.sync_copy(sin_buf, sin_out.at[pl.ds(base + j * _CHUNK, _CHUNK)])


@jax.jit
def _gather(cos_tab, sin_tab, idx):
    mesh = plsc.VectorSubcoreMesh(core_axis_name="c", subcore_axis_name="s")
    f = pl.kernel(
        _gather_body,
        mesh=mesh,
        out_type=(
            jax.ShapeDtypeStruct((_B, DIM), jnp.float32),
            jax.ShapeDtypeStruct((_B, DIM), jnp.float32),
        ),
        scratch_types=[
            pltpu.VMEM((_NCHUNK, _CHUNK), jnp.int32),
            pltpu.VMEM((_CHUNK, DIM), jnp.float32),
            pltpu.VMEM((_CHUNK, DIM), jnp.float32),
            pltpu.SemaphoreType.DMA,
            pltpu.SemaphoreType.DMA,
        ],
    )
    return f(cos_tab, sin_tab, idx)


def kernel(x, position_ids):
    cos_tab = jnp.asarray(_COS_TAB)
    sin_tab = jnp.asarray(_SIN_TAB)
    idx = position_ids.reshape(_NW * _NCHUNK, _CHUNK)
    cos_rows, sin_rows = _gather(cos_tab, sin_tab, idx)
    b, s = position_ids.shape
    return (cos_rows.reshape(b, s, DIM).astype(x.dtype),
            sin_rows.reshape(b, s, DIM).astype(x.dtype))
